# Initial kernel scaffold; baseline (speedup 1.0000x reference)
import jax
import jax.numpy as jnp
from jax.experimental import pallas as pl


def kernel(x, edge_index, W, b, gamma, beta):
    raise NotImplementedError("write your pallas kernel here")


# speedup vs baseline: 7.3117x; 7.3117x over previous
import functools

import jax
import jax.numpy as jnp
from jax import lax
from jax.experimental import pallas as pl
from jax.experimental.pallas import tpu as pltpu
from jax.experimental.pallas import tpu_sc as plsc

N = 10000
E = 320000
D = 128

NC = 2
NS = 16
NW = NC * NS

NP = 10240
EP = NW * NP
EPW = EP // NW
CB = 128
NCHUNK = EPW // CB
ROWS = NP // 128
NPT = NP // NS

_mesh = plsc.VectorSubcoreMesh(core_axis_name="c", subcore_axis_name="s")


@functools.partial(
    pl.kernel,
    out_type=jax.ShapeDtypeStruct((NC, 2, NP), jnp.float32),
    mesh=_mesh,
    compiler_params=pltpu.CompilerParams(needs_layout_passes=False),
    scratch_types=[
        pltpu.VMEM((ROWS, CB), jnp.int32),
        pltpu.VMEM((ROWS, CB), jnp.int32),
        pltpu.VMEM((NP,), jnp.float32),
        pltpu.VMEM((NP,), jnp.float32),
        pltpu.VMEM((2, NPT), jnp.float32),
        pltpu.VMEM((2, NPT), jnp.float32),
        pltpu.VMEM_SHARED((NS, 2, NP), jnp.float32),
    ],
)
def _deg_kernel(src_hbm, dst_hbm, out_hbm,
                sidx, didx, hist_s, hist_d, acc, tmp, sh_all):
  cid = lax.axis_index("c")
  sid = lax.axis_index("s")
  wid = sid * NC + cid

  zeros16 = jnp.zeros((16,), jnp.float32)
  ones16 = jnp.ones((16,), jnp.float32)

  def zbody(i, carry):
    hist_s[pl.ds(i * 16, 16)] = zeros16
    hist_d[pl.ds(i * 16, 16)] = zeros16
    return carry

  lax.fori_loop(0, NP // 16, zbody, 0)

  pltpu.sync_copy(src_hbm.at[pl.ds(wid * ROWS, ROWS)], sidx)
  pltpu.sync_copy(dst_hbm.at[pl.ds(wid * ROWS, ROWS)], didx)

  def hbody(i, carry):
    r = i >> 3
    c = (i & 7) << 4
    vs = sidx[r, pl.ds(c, 16)]
    plsc.addupdate_scatter(hist_s, [vs], ones16)
    vd = didx[r, pl.ds(c, 16)]
    plsc.addupdate_scatter(hist_d, [vd], ones16)
    return carry

  lax.fori_loop(0, ROWS * 8, hbody, 0)

  pltpu.sync_copy(hist_s, sh_all.at[sid, 0])
  pltpu.sync_copy(hist_d, sh_all.at[sid, 1])
  plsc.subcore_barrier()

  base = sid * NPT
  pltpu.sync_copy(sh_all.at[0, :, pl.ds(base, NPT)], acc)
  for t in range(1, NS):
    pltpu.sync_copy(sh_all.at[t, :, pl.ds(base, NPT)], tmp)
    for h in range(2):
      def abody(j, carry):
        acc[h, pl.ds(j * 16, 16)] = (
            acc[h, pl.ds(j * 16, 16)] + tmp[h, pl.ds(j * 16, 16)])
        return carry
      lax.fori_loop(0, NPT // 16, abody, 0)

  pltpu.sync_copy(acc.at[0], out_hbm.at[cid, 0, pl.ds(base, NPT)])
  pltpu.sync_copy(acc.at[1], out_hbm.at[cid, 1, pl.ds(base, NPT)])


@functools.partial(
    pl.kernel,
    out_type=jax.ShapeDtypeStruct((NC, NP, D), jnp.float32),
    mesh=_mesh,
    scratch_types=[
        pltpu.VMEM((NCHUNK, CB), jnp.int32),
        pltpu.VMEM((NCHUNK, CB), jnp.int32),
        pltpu.VMEM((CB, D), jnp.float32),
        pltpu.VMEM_SHARED((NP, D), jnp.float32),
        pltpu.SemaphoreType.DMA,
    ],
)
def _gs_kernel(h_hbm, src_hbm, dst_hbm, zeros_hbm, out_hbm,
               sidx, didx, buf, agg_sh, sem):
  cid = lax.axis_index("c")
  sid = lax.axis_index("s")
  wid = sid * NC + cid

  pltpu.sync_copy(zeros_hbm, agg_sh.at[pl.ds(sid * NPT, NPT)])
  pltpu.sync_copy(src_hbm.at[pl.ds(wid * NCHUNK, NCHUNK)], sidx)
  pltpu.sync_copy(dst_hbm.at[pl.ds(wid * NCHUNK, NCHUNK)], didx)
  plsc.subcore_barrier()

  def body(j, carry):
    pltpu.async_copy(h_hbm.at[sidx.at[j]], buf, sem).wait()
    pltpu.sync_copy(buf, agg_sh.at[didx.at[j]], add=True)
    return carry

  lax.fori_loop(0, NCHUNK, body, 0)

  plsc.subcore_barrier()
  pltpu.sync_copy(agg_sh.at[pl.ds(sid * NPT, NPT)],
                  out_hbm.at[cid, pl.ds(sid * NPT, NPT)])


def _mm_body(x_ref, w_ref, ns_ref, o_ref):
  h = jnp.dot(x_ref[...], w_ref[...], preferred_element_type=jnp.float32)
  o_ref[...] = h * ns_ref[...]


_BN = 1024
_NBLK = NP // _BN


def _mm_call(x_pad, W, ns):
  return pl.pallas_call(
      _mm_body,
      grid=(_NBLK,),
      in_specs=[
          pl.BlockSpec((_BN, D), lambda i: (i, 0)),
          pl.BlockSpec((D, D), lambda i: (0, 0)),
          pl.BlockSpec((_BN, 1), lambda i: (i, 0)),
      ],
      out_specs=pl.BlockSpec((_BN, D), lambda i: (i, 0)),
      out_shape=jax.ShapeDtypeStruct((NP, D), jnp.float32),
  )(x_pad, W, ns)


def _comb_body(p0_ref, p1_ref, nd_ref, b_ref, h2_ref, st_ref):
  i = pl.program_id(0)
  h = (p0_ref[...] + p1_ref[...]) * nd_ref[...] + b_ref[...]
  h2_ref[...] = h
  rows = lax.broadcasted_iota(jnp.int32, (_BN, D), 0) + i * _BN
  hm = jnp.where(rows < N, h, 0.0)
  s1 = jnp.sum(hm, axis=0, keepdims=True)
  s2 = jnp.sum(hm * hm, axis=0, keepdims=True)
  acc = jnp.concatenate(
      [s1, s2, jnp.zeros((6, D), jnp.float32)], axis=0)

  @pl.when(i == 0)
  def _():
    st_ref[...] = acc

  @pl.when(i > 0)
  def _():
    st_ref[...] = st_ref[...] + acc


def _comb_call(p0, p1, nd, b2):
  return pl.pallas_call(
      _comb_body,
      grid=(_NBLK,),
      in_specs=[
          pl.BlockSpec((_BN, D), lambda i: (i, 0)),
          pl.BlockSpec((_BN, D), lambda i: (i, 0)),
          pl.BlockSpec((_BN, 1), lambda i: (i, 0)),
          pl.BlockSpec((1, D), lambda i: (0, 0)),
      ],
      out_specs=[
          pl.BlockSpec((_BN, D), lambda i: (i, 0)),
          pl.BlockSpec((8, D), lambda i: (0, 0)),
      ],
      out_shape=[
          jax.ShapeDtypeStruct((NP, D), jnp.float32),
          jax.ShapeDtypeStruct((8, D), jnp.float32),
      ],
  )(p0, p1, nd, b2)


def _bn_body(h2_ref, st_ref, g_ref, be_ref, x_ref, o_ref):
  mean = st_ref[0:1, :] * (1.0 / N)
  ex2 = st_ref[1:2, :] * (1.0 / N)
  var = ex2 - mean * mean
  inv = lax.rsqrt(var + 1e-5)
  hn = (h2_ref[...] - mean) * inv * g_ref[...] + be_ref[...]
  out = jnp.where(hn > 0, hn, jnp.exp(jnp.minimum(hn, 0.0)) - 1.0)
  o_ref[...] = x_ref[...] + out


def _bn_call(h2, stats, g2, be2, x_pad):
  return pl.pallas_call(
      _bn_body,
      grid=(_NBLK,),
      in_specs=[
          pl.BlockSpec((_BN, D), lambda i: (i, 0)),
          pl.BlockSpec((8, D), lambda i: (0, 0)),
          pl.BlockSpec((1, D), lambda i: (0, 0)),
          pl.BlockSpec((1, D), lambda i: (0, 0)),
          pl.BlockSpec((_BN, D), lambda i: (i, 0)),
      ],
      out_specs=pl.BlockSpec((_BN, D), lambda i: (i, 0)),
      out_shape=jax.ShapeDtypeStruct((NP, D), jnp.float32),
  )(h2, stats, g2, be2, x_pad)


@jax.jit
def kernel(x, edge_index, W, b, gamma, beta):
  src = edge_index[0]
  dst = edge_index[1]
  pad = jnp.full((EP - E,), N, jnp.int32)
  src_p = jnp.concatenate([src, pad]).reshape(EP // CB, CB)
  dst_p = jnp.concatenate([dst, pad]).reshape(EP // CB, CB)
  degp = _deg_kernel(src_p, dst_p)
  deg = degp[0] + degp[1]
  ns = lax.rsqrt(jnp.maximum(deg[0], 1.0))[:, None]
  nd = lax.rsqrt(jnp.maximum(deg[1], 1.0))[:, None]

  x_pad = jnp.concatenate(
      [x, jnp.zeros((NP - N, D), jnp.float32)], axis=0)
  h_scaled = _mm_call(x_pad, W, ns)

  zeros_blk = jnp.zeros((NPT, D), jnp.float32)
  parts = _gs_kernel(h_scaled, src_p, dst_p, zeros_blk)

  h2, stats = _comb_call(parts[0], parts[1], nd, b.reshape(1, D))
  out_pad = _bn_call(h2, stats, gamma.reshape(1, D),
                     beta.reshape(1, D), x_pad)
  return out_pad[:N]


# speedup vs baseline: 9.0847x; 1.2425x over previous
import functools

import jax
import jax.numpy as jnp
from jax import lax
from jax.experimental import pallas as pl
from jax.experimental.pallas import tpu as pltpu
from jax.experimental.pallas import tpu_sc as plsc

N = 10000
E = 320000
D = 128

NC = 2
NS = 16
NW = NC * NS

NP = 10240
EP = NW * NP
EPW = EP // NW
CB = 128
NCHUNK = EPW // CB
NCH_F = 117
NCH_S = 2 * NCHUNK - NCH_F
FAST_CID = 0
ROWS = NP // 128
NPT = NP // NS

_mesh = plsc.VectorSubcoreMesh(core_axis_name="c", subcore_axis_name="s")


@functools.partial(
    pl.kernel,
    out_type=jax.ShapeDtypeStruct((NC, NS, 2, NP), jnp.float32),
    mesh=_mesh,
    compiler_params=pltpu.CompilerParams(needs_layout_passes=False),
    scratch_types=[
        pltpu.VMEM((EPW,), jnp.int32),
        pltpu.VMEM((NP,), jnp.float32),
        pltpu.VMEM((NP,), jnp.float32),
    ],
)
def _deg_kernel(sd_hbm, out_hbm, sdidx, hist_s, hist_d):
  cid = lax.axis_index("c")
  sid = lax.axis_index("s")
  wid = sid * NC + cid

  zeros16 = jnp.zeros((16,), jnp.float32)
  ones16 = jnp.ones((16,), jnp.float32)

  def zbody(i, carry):
    hist_s[pl.ds(i * 16, 16)] = zeros16
    hist_d[pl.ds(i * 16, 16)] = zeros16
    return carry

  lax.fori_loop(0, NP // 16, zbody, 0)

  pltpu.sync_copy(sd_hbm.at[pl.ds(wid * EPW, EPW)], sdidx)

  def hbody(i, carry):
    v = sdidx[pl.ds(i * 16, 16)]
    plsc.addupdate_scatter(hist_s, [v & 0xFFFF], ones16)
    plsc.addupdate_scatter(hist_d, [v >> 16], ones16)
    return carry

  lax.fori_loop(0, EPW // 16, hbody, 0)

  pltpu.sync_copy(hist_s, out_hbm.at[cid, sid, 0])
  pltpu.sync_copy(hist_d, out_hbm.at[cid, sid, 1])


@functools.partial(
    pl.kernel,
    out_type=jax.ShapeDtypeStruct((NC, NP, D), jnp.float32),
    mesh=_mesh,
    scratch_types=[
        pltpu.VMEM((CB,), jnp.int32),
        pltpu.VMEM((CB,), jnp.int32),
        pltpu.VMEM((CB,), jnp.int32),
        pltpu.VMEM((CB, D), jnp.float32),
        pltpu.VMEM_SHARED((NP, D), jnp.float32),
        pltpu.SemaphoreType.DMA,
    ],
)
def _gs_kernel(h_hbm, sd_hbm, out_hbm,
               sdc, sic, dic, buf0, agg_sh, sem0):
  cid = lax.axis_index("c")
  sid = lax.axis_index("s")

  is_fast = cid == FAST_CID
  nch = jnp.where(is_fast, NCH_F, NCH_S)
  base = jnp.where(is_fast, sid * NCH_F, NS * NCH_F + sid * NCH_S)

  pltpu.sync_copy(h_hbm.at[pl.ds(NP - CB, CB)], buf0)
  for k in range(NPT // CB):
    pltpu.sync_copy(buf0, agg_sh.at[pl.ds(sid * NPT + k * CB, CB)])
  plsc.subcore_barrier()

  def body(j, carry):
    pltpu.sync_copy(sd_hbm.at[pl.ds((base + j) * CB, CB)], sdc)

    def ubody(i, carry2):
      v = sdc[pl.ds(i * 16, 16)]
      sic[pl.ds(i * 16, 16)] = v & 0xFFFF
      dic[pl.ds(i * 16, 16)] = v >> 16
      return carry2

    lax.fori_loop(0, CB // 16, ubody, 0)
    pltpu.async_copy(h_hbm.at[sic], buf0, sem0).wait()
    pltpu.sync_copy(buf0, agg_sh.at[dic], add=True)
    return carry

  lax.fori_loop(0, nch, body, 0)

  plsc.subcore_barrier()
  pltpu.sync_copy(agg_sh.at[pl.ds(sid * NPT, NPT)],
                  out_hbm.at[cid, pl.ds(sid * NPT, NPT)])


def _nrm_body(dp_ref, o_ref):
  x = dp_ref[...]
  row = lax.broadcasted_iota(jnp.int32, (2 * NW, _BN), 0)
  even = (row & 1) == 0
  s_src = jnp.sum(jnp.where(even, x, 0.0), axis=0, keepdims=True)
  s_dst = jnp.sum(jnp.where(even, 0.0, x), axis=0, keepdims=True)
  deg = jnp.concatenate([s_src, s_dst], axis=0)
  o_ref[...] = lax.rsqrt(jnp.maximum(deg, 1.0))


def _nrm_call(degp64):
  return pl.pallas_call(
      _nrm_body,
      grid=(NP // _BN,),
      in_specs=[pl.BlockSpec((2 * NW, _BN), lambda i: (0, i))],
      out_specs=pl.BlockSpec((2, _BN), lambda i: (0, i)),
      out_shape=jax.ShapeDtypeStruct((2, NP), jnp.float32),
  )(degp64)


def _mm_body(x_ref, w_ref, ns_ref, o_ref):
  h = jnp.dot(x_ref[...], w_ref[...], preferred_element_type=jnp.float32)
  o_ref[...] = h * ns_ref[...]


_BN = 1024
_NBLK = NP // _BN


def _mm_call(x_pad, W, ns):
  return pl.pallas_call(
      _mm_body,
      grid=(_NBLK,),
      in_specs=[
          pl.BlockSpec((_BN, D), lambda i: (i, 0)),
          pl.BlockSpec((D, D), lambda i: (0, 0)),
          pl.BlockSpec((_BN, 1), lambda i: (i, 0)),
      ],
      out_specs=pl.BlockSpec((_BN, D), lambda i: (i, 0)),
      out_shape=jax.ShapeDtypeStruct((NP, D), jnp.float32),
  )(x_pad, W, ns)


def _comb_body(p0_ref, p1_ref, nd_ref, b_ref, h2_ref, st_ref):
  i = pl.program_id(0)
  h = (p0_ref[...] + p1_ref[...]) * nd_ref[...] + b_ref[...]
  h2_ref[...] = h
  rows = lax.broadcasted_iota(jnp.int32, (_BN, D), 0) + i * _BN
  hm = jnp.where(rows < N, h, 0.0)
  s1 = jnp.sum(hm, axis=0, keepdims=True)
  s2 = jnp.sum(hm * hm, axis=0, keepdims=True)
  acc = jnp.concatenate(
      [s1, s2, jnp.zeros((6, D), jnp.float32)], axis=0)

  @pl.when(i == 0)
  def _():
    st_ref[...] = acc

  @pl.when(i > 0)
  def _():
    st_ref[...] = st_ref[...] + acc


def _comb_call(p0, p1, nd, b2):
  return pl.pallas_call(
      _comb_body,
      grid=(_NBLK,),
      in_specs=[
          pl.BlockSpec((_BN, D), lambda i: (i, 0)),
          pl.BlockSpec((_BN, D), lambda i: (i, 0)),
          pl.BlockSpec((_BN, 1), lambda i: (i, 0)),
          pl.BlockSpec((1, D), lambda i: (0, 0)),
      ],
      out_specs=[
          pl.BlockSpec((_BN, D), lambda i: (i, 0)),
          pl.BlockSpec((8, D), lambda i: (0, 0)),
      ],
      out_shape=[
          jax.ShapeDtypeStruct((NP, D), jnp.float32),
          jax.ShapeDtypeStruct((8, D), jnp.float32),
      ],
  )(p0, p1, nd, b2)


def _bn_body(h2_ref, st_ref, g_ref, be_ref, x_ref, o_ref):
  mean = st_ref[0:1, :] * (1.0 / N)
  ex2 = st_ref[1:2, :] * (1.0 / N)
  var = ex2 - mean * mean
  inv = lax.rsqrt(var + 1e-5)
  hn = (h2_ref[...] - mean) * inv * g_ref[...] + be_ref[...]
  out = jnp.where(hn > 0, hn, jnp.exp(jnp.minimum(hn, 0.0)) - 1.0)
  o_ref[...] = x_ref[...] + out


def _bn_call(h2, stats, g2, be2, x_pad):
  return pl.pallas_call(
      _bn_body,
      grid=(_NBLK,),
      in_specs=[
          pl.BlockSpec((_BN, D), lambda i: (i, 0)),
          pl.BlockSpec((8, D), lambda i: (0, 0)),
          pl.BlockSpec((1, D), lambda i: (0, 0)),
          pl.BlockSpec((1, D), lambda i: (0, 0)),
          pl.BlockSpec((_BN, D), lambda i: (i, 0)),
      ],
      out_specs=pl.BlockSpec((_BN, D), lambda i: (i, 0)),
      out_shape=jax.ShapeDtypeStruct((NP, D), jnp.float32),
  )(h2, stats, g2, be2, x_pad)


@jax.jit
def kernel(x, edge_index, W, b, gamma, beta):
  src = edge_index[0]
  dst = edge_index[1]
  pad = jnp.full((EP - E,), N, jnp.int32)
  src_p = jnp.concatenate([src, pad])
  dst_p = jnp.concatenate([dst, pad])
  sd_p = src_p | (dst_p << 16)
  degp = _deg_kernel(sd_p)
  norms = lax.rsqrt(jnp.maximum(degp.sum(axis=(0, 1)), 1.0))
  ns = norms[0][:, None]
  nd = norms[1][:, None]

  x_pad = jnp.concatenate(
      [x, jnp.zeros((NP - N, D), jnp.float32)], axis=0)
  h_scaled = _mm_call(x_pad, W, ns)

  parts = _gs_kernel(h_scaled, sd_p)

  h2, stats = _comb_call(parts[0], parts[1], nd, b.reshape(1, D))
  out_pad = _bn_call(h2, stats, gamma.reshape(1, D),
                     beta.reshape(1, D), x_pad)
  return out_pad[:N]


# speedup vs baseline: 9.1151x; 1.0034x over previous
import functools

import jax
import jax.numpy as jnp
from jax import lax
from jax.experimental import pallas as pl
from jax.experimental.pallas import tpu as pltpu
from jax.experimental.pallas import tpu_sc as plsc

N = 10000
E = 320000
D = 128

NC = 2
NS = 16
NW = NC * NS

NP = 10240
EP = NW * NP
EPW = EP // NW
CB = 128
NCHUNK = EPW // CB
NCH_F = 117
NCH_S = 2 * NCHUNK - NCH_F
FAST_CID = 0
ROWS = NP // 128
NPT = NP // NS

_mesh = plsc.VectorSubcoreMesh(core_axis_name="c", subcore_axis_name="s")


@functools.partial(
    pl.kernel,
    out_type=jax.ShapeDtypeStruct((NC, NS, 2, NP), jnp.float32),
    mesh=_mesh,
    compiler_params=pltpu.CompilerParams(needs_layout_passes=False),
    scratch_types=[
        pltpu.VMEM((EPW,), jnp.int32),
        pltpu.VMEM((NP,), jnp.float32),
        pltpu.VMEM((NP,), jnp.float32),
    ],
)
def _deg_kernel(sd_hbm, out_hbm, sdidx, hist_s, hist_d):
  cid = lax.axis_index("c")
  sid = lax.axis_index("s")
  wid = sid * NC + cid

  zeros16 = jnp.zeros((16,), jnp.float32)
  ones16 = jnp.ones((16,), jnp.float32)

  def zbody(i, carry):
    hist_s[pl.ds(i * 16, 16)] = zeros16
    hist_d[pl.ds(i * 16, 16)] = zeros16
    return carry

  lax.fori_loop(0, NP // 16, zbody, 0)

  pltpu.sync_copy(sd_hbm.at[pl.ds(wid * EPW, EPW)], sdidx)

  def hbody(i, carry):
    v = sdidx[pl.ds(i * 16, 16)]
    plsc.addupdate_scatter(hist_s, [v & 0xFFFF], ones16)
    plsc.addupdate_scatter(hist_d, [v >> 16], ones16)
    return carry

  lax.fori_loop(0, EPW // 16, hbody, 0)

  pltpu.sync_copy(hist_s, out_hbm.at[cid, sid, 0])
  pltpu.sync_copy(hist_d, out_hbm.at[cid, sid, 1])


@functools.partial(
    pl.kernel,
    out_type=jax.ShapeDtypeStruct((NC, NP, D), jnp.float32),
    mesh=_mesh,
    scratch_types=[
        pltpu.VMEM((NCH_F * CB,), jnp.int32),
        pltpu.VMEM((NCH_F * CB,), jnp.int32),
        pltpu.VMEM((CB, D), jnp.float32),
        pltpu.VMEM_SHARED((NP, D), jnp.float32),
        pltpu.SemaphoreType.DMA,
    ],
)
def _gs_kernel(h_hbm, sd_hbm, out_hbm,
               sidx, didx, buf0, agg_sh, sem0):
  cid = lax.axis_index("c")
  sid = lax.axis_index("s")

  is_fast = cid == FAST_CID
  nch = jnp.where(is_fast, NCH_F, NCH_S)
  base = jnp.where(is_fast, sid * NCH_F, NS * NCH_F + sid * NCH_S)

  pltpu.sync_copy(h_hbm.at[pl.ds(NP - CB, CB)], buf0)
  for k in range(NPT // CB):
    pltpu.sync_copy(buf0, agg_sh.at[pl.ds(sid * NPT + k * CB, CB)])
  ld_base = jnp.minimum(base * CB, EP - NCH_F * CB)
  off = base * CB - ld_base
  pltpu.sync_copy(sd_hbm.at[pl.ds(ld_base, NCH_F * CB)], sidx)

  def ubody(i, carry):
    v = sidx[pl.ds(i * 16, 16)]
    sidx[pl.ds(i * 16, 16)] = v & 0xFFFF
    didx[pl.ds(i * 16, 16)] = v >> 16
    return carry

  lax.fori_loop(0, NCH_F * CB // 16, ubody, 0)
  plsc.subcore_barrier()

  def body(j, carry):
    s = off + j * CB
    pltpu.async_copy(h_hbm.at[sidx.at[pl.ds(s, CB)]], buf0, sem0).wait()
    pltpu.sync_copy(buf0, agg_sh.at[didx.at[pl.ds(s, CB)]], add=True)
    return carry

  lax.fori_loop(0, nch, body, 0)

  plsc.subcore_barrier()
  pltpu.sync_copy(agg_sh.at[pl.ds(sid * NPT, NPT)],
                  out_hbm.at[cid, pl.ds(sid * NPT, NPT)])


def _nrm_body(dp_ref, o_ref):
  x = dp_ref[...]
  row = lax.broadcasted_iota(jnp.int32, (2 * NW, _BN), 0)
  even = (row & 1) == 0
  s_src = jnp.sum(jnp.where(even, x, 0.0), axis=0, keepdims=True)
  s_dst = jnp.sum(jnp.where(even, 0.0, x), axis=0, keepdims=True)
  deg = jnp.concatenate([s_src, s_dst], axis=0)
  o_ref[...] = lax.rsqrt(jnp.maximum(deg, 1.0))


def _nrm_call(degp64):
  return pl.pallas_call(
      _nrm_body,
      grid=(NP // _BN,),
      in_specs=[pl.BlockSpec((2 * NW, _BN), lambda i: (0, i))],
      out_specs=pl.BlockSpec((2, _BN), lambda i: (0, i)),
      out_shape=jax.ShapeDtypeStruct((2, NP), jnp.float32),
  )(degp64)


def _mm_body(x_ref, w_ref, ns_ref, o_ref):
  h = jnp.dot(x_ref[...], w_ref[...], preferred_element_type=jnp.float32)
  o_ref[...] = h * ns_ref[...]


_BN = 1024
_NBLK = NP // _BN


def _mm_call(x_pad, W, ns):
  return pl.pallas_call(
      _mm_body,
      grid=(_NBLK,),
      in_specs=[
          pl.BlockSpec((_BN, D), lambda i: (i, 0)),
          pl.BlockSpec((D, D), lambda i: (0, 0)),
          pl.BlockSpec((_BN, 1), lambda i: (i, 0)),
      ],
      out_specs=pl.BlockSpec((_BN, D), lambda i: (i, 0)),
      out_shape=jax.ShapeDtypeStruct((NP, D), jnp.float32),
  )(x_pad, W, ns)


def _comb_body(p0_ref, p1_ref, nd_ref, b_ref, h2_ref, st_ref):
  i = pl.program_id(0)
  h = (p0_ref[...] + p1_ref[...]) * nd_ref[...] + b_ref[...]
  h2_ref[...] = h
  rows = lax.broadcasted_iota(jnp.int32, (_BN, D), 0) + i * _BN
  hm = jnp.where(rows < N, h, 0.0)
  s1 = jnp.sum(hm, axis=0, keepdims=True)
  s2 = jnp.sum(hm * hm, axis=0, keepdims=True)
  acc = jnp.concatenate(
      [s1, s2, jnp.zeros((6, D), jnp.float32)], axis=0)

  @pl.when(i == 0)
  def _():
    st_ref[...] = acc

  @pl.when(i > 0)
  def _():
    st_ref[...] = st_ref[...] + acc


def _comb_call(p0, p1, nd, b2):
  return pl.pallas_call(
      _comb_body,
      grid=(_NBLK,),
      in_specs=[
          pl.BlockSpec((_BN, D), lambda i: (i, 0)),
          pl.BlockSpec((_BN, D), lambda i: (i, 0)),
          pl.BlockSpec((_BN, 1), lambda i: (i, 0)),
          pl.BlockSpec((1, D), lambda i: (0, 0)),
      ],
      out_specs=[
          pl.BlockSpec((_BN, D), lambda i: (i, 0)),
          pl.BlockSpec((8, D), lambda i: (0, 0)),
      ],
      out_shape=[
          jax.ShapeDtypeStruct((NP, D), jnp.float32),
          jax.ShapeDtypeStruct((8, D), jnp.float32),
      ],
  )(p0, p1, nd, b2)


def _bn_body(h2_ref, st_ref, g_ref, be_ref, x_ref, o_ref):
  mean = st_ref[0:1, :] * (1.0 / N)
  ex2 = st_ref[1:2, :] * (1.0 / N)
  var = ex2 - mean * mean
  inv = lax.rsqrt(var + 1e-5)
  hn = (h2_ref[...] - mean) * inv * g_ref[...] + be_ref[...]
  out = jnp.where(hn > 0, hn, jnp.exp(jnp.minimum(hn, 0.0)) - 1.0)
  o_ref[...] = x_ref[...] + out


def _bn_call(h2, stats, g2, be2, x_pad):
  return pl.pallas_call(
      _bn_body,
      grid=(_NBLK,),
      in_specs=[
          pl.BlockSpec((_BN, D), lambda i: (i, 0)),
          pl.BlockSpec((8, D), lambda i: (0, 0)),
          pl.BlockSpec((1, D), lambda i: (0, 0)),
          pl.BlockSpec((1, D), lambda i: (0, 0)),
          pl.BlockSpec((_BN, D), lambda i: (i, 0)),
      ],
      out_specs=pl.BlockSpec((_BN, D), lambda i: (i, 0)),
      out_shape=jax.ShapeDtypeStruct((NP, D), jnp.float32),
  )(h2, stats, g2, be2, x_pad)


@jax.jit
def kernel(x, edge_index, W, b, gamma, beta):
  src = edge_index[0]
  dst = edge_index[1]
  pad = jnp.full((EP - E,), N, jnp.int32)
  src_p = jnp.concatenate([src, pad])
  dst_p = jnp.concatenate([dst, pad])
  sd_p = src_p | (dst_p << 16)
  degp = _deg_kernel(sd_p)
  norms = lax.rsqrt(jnp.maximum(degp.sum(axis=(0, 1)), 1.0))
  ns = norms[0][:, None]
  nd = norms[1][:, None]

  x_pad = jnp.concatenate(
      [x, jnp.zeros((NP - N, D), jnp.float32)], axis=0)
  h_scaled = _mm_call(x_pad, W, ns)

  parts = _gs_kernel(h_scaled, sd_p)

  h2, stats = _comb_call(parts[0], parts[1], nd, b.reshape(1, D))
  out_pad = _bn_call(h2, stats, gamma.reshape(1, D),
                     beta.reshape(1, D), x_pad)
  return out_pad[:N]


# speedup vs baseline: 9.7023x; 1.0644x over previous
import functools

import jax
import jax.numpy as jnp
from jax import lax
from jax.experimental import pallas as pl
from jax.experimental.pallas import tpu as pltpu
from jax.experimental.pallas import tpu_sc as plsc

N = 10000
E = 320000
D = 128

NC = 2
NS = 16
NW = NC * NS

NP = 10240
EP = NW * NP
EPW = EP // NW
CB = 128
NCHUNK = EPW // CB
NCH_F = 129
NCH_S = 2 * NCHUNK - NCH_F
FAST_CID = 0
DGF = 960
DGS = 2 * (EPW // 16) - DGF
AGR = 10112
AGT = AGR // NS
ROWS = NP // 128
NPT = NP // NS

_mesh = plsc.VectorSubcoreMesh(core_axis_name="c", subcore_axis_name="s")


@functools.partial(
    pl.kernel,
    out_type=jax.ShapeDtypeStruct((NC, NS, 2, NP), jnp.float32),
    mesh=_mesh,
    compiler_params=pltpu.CompilerParams(needs_layout_passes=False),
    scratch_types=[
        pltpu.VMEM((DGF * 16,), jnp.int32),
        pltpu.VMEM((NP,), jnp.float32),
        pltpu.VMEM((NP,), jnp.float32),
    ],
)
def _deg_kernel(sd_hbm, out_hbm, sdidx, hist_s, hist_d):
  cid = lax.axis_index("c")
  sid = lax.axis_index("s")

  is_fast = cid == FAST_CID
  ngr = jnp.where(is_fast, DGF, DGS)
  base_el = jnp.where(is_fast, sid * DGF * 16,
                      NS * DGF * 16 + sid * DGS * 16)
  ld_base = jnp.minimum(base_el, EP - DGF * 16)
  off = base_el - ld_base

  zeros16 = jnp.zeros((16,), jnp.float32)
  ones16 = jnp.ones((16,), jnp.float32)

  def zbody(i, carry):
    hist_s[pl.ds(i * 16, 16)] = zeros16
    hist_d[pl.ds(i * 16, 16)] = zeros16
    return carry

  lax.fori_loop(0, NP // 16, zbody, 0)

  pltpu.sync_copy(sd_hbm.at[pl.ds(ld_base, DGF * 16)], sdidx)

  def hbody(i, carry):
    v = sdidx[pl.ds(off + i * 16, 16)]
    plsc.addupdate_scatter(hist_s, [v & 0xFFFF], ones16)
    plsc.addupdate_scatter(hist_d, [v >> 16], ones16)
    return carry

  lax.fori_loop(0, ngr, hbody, 0)

  pltpu.sync_copy(hist_s, out_hbm.at[cid, sid, 0])
  pltpu.sync_copy(hist_d, out_hbm.at[cid, sid, 1])


@functools.partial(
    pl.kernel,
    out_type=jax.ShapeDtypeStruct((NC, NP, D), jnp.float32),
    mesh=_mesh,
    scratch_types=[
        pltpu.VMEM((NCH_F * CB,), jnp.int32),
        pltpu.VMEM((NCH_F * CB,), jnp.int32),
        pltpu.VMEM((CB, D), jnp.float32),
        pltpu.VMEM_SHARED((AGR, D), jnp.float32),
        pltpu.SemaphoreType.DMA,
    ],
)
def _gs_kernel(h_hbm, sd_hbm, out_hbm,
               sidx, didx, buf0, agg_sh, sem0):
  cid = lax.axis_index("c")
  sid = lax.axis_index("s")

  is_fast = cid == FAST_CID
  nch = jnp.where(is_fast, NCH_F, NCH_S)
  base = jnp.where(is_fast, sid * NCH_F, NS * NCH_F + sid * NCH_S)

  pltpu.sync_copy(h_hbm.at[pl.ds(NP - CB, CB)], buf0)
  for k in range(AGT // CB):
    pltpu.sync_copy(buf0, agg_sh.at[pl.ds(sid * AGT + k * CB, CB)])
  rem = AGT - (AGT // CB) * CB
  pltpu.sync_copy(buf0.at[pl.ds(0, rem)],
                  agg_sh.at[pl.ds(sid * AGT + (AGT // CB) * CB, rem)])
  ld_base = jnp.minimum(base * CB, EP - NCH_F * CB)
  off = base * CB - ld_base
  pltpu.sync_copy(sd_hbm.at[pl.ds(ld_base, NCH_F * CB)], sidx)

  def ubody(i, carry):
    v = sidx[pl.ds(i * 16, 16)]
    sidx[pl.ds(i * 16, 16)] = v & 0xFFFF
    didx[pl.ds(i * 16, 16)] = v >> 16
    return carry

  lax.fori_loop(0, NCH_F * CB // 16, ubody, 0)
  plsc.subcore_barrier()

  def body(j, carry):
    s = off + j * CB
    pltpu.async_copy(h_hbm.at[sidx.at[pl.ds(s, CB)]], buf0, sem0).wait()
    pltpu.sync_copy(buf0, agg_sh.at[didx.at[pl.ds(s, CB)]], add=True)
    return carry

  lax.fori_loop(0, nch, body, 0)

  plsc.subcore_barrier()
  pltpu.sync_copy(agg_sh.at[pl.ds(sid * AGT, AGT)],
                  out_hbm.at[cid, pl.ds(sid * AGT, AGT)])


def _nrm_body(dp_ref, o_ref):
  x = dp_ref[...]
  row = lax.broadcasted_iota(jnp.int32, (2 * NW, _BN), 0)
  even = (row & 1) == 0
  s_src = jnp.sum(jnp.where(even, x, 0.0), axis=0, keepdims=True)
  s_dst = jnp.sum(jnp.where(even, 0.0, x), axis=0, keepdims=True)
  deg = jnp.concatenate([s_src, s_dst], axis=0)
  o_ref[...] = lax.rsqrt(jnp.maximum(deg, 1.0))


def _nrm_call(degp64):
  return pl.pallas_call(
      _nrm_body,
      grid=(NP // _BN,),
      in_specs=[pl.BlockSpec((2 * NW, _BN), lambda i: (0, i))],
      out_specs=pl.BlockSpec((2, _BN), lambda i: (0, i)),
      out_shape=jax.ShapeDtypeStruct((2, NP), jnp.float32),
  )(degp64)


def _mm_body(x_ref, w_ref, ns_ref, o_ref):
  h = jnp.dot(x_ref[...], w_ref[...], preferred_element_type=jnp.float32)
  o_ref[...] = h * ns_ref[...]


_BN = 1024
_NBLK = NP // _BN


def _mm_call(x_pad, W, ns):
  return pl.pallas_call(
      _mm_body,
      grid=(_NBLK,),
      in_specs=[
          pl.BlockSpec((_BN, D), lambda i: (i, 0)),
          pl.BlockSpec((D, D), lambda i: (0, 0)),
          pl.BlockSpec((_BN, 1), lambda i: (i, 0)),
      ],
      out_specs=pl.BlockSpec((_BN, D), lambda i: (i, 0)),
      out_shape=jax.ShapeDtypeStruct((NP, D), jnp.float32),
  )(x_pad, W, ns)


def _comb_body(p0_ref, p1_ref, nd_ref, b_ref, h2_ref, st_ref):
  i = pl.program_id(0)
  h = (p0_ref[...] + p1_ref[...]) * nd_ref[...] + b_ref[...]
  h2_ref[...] = h
  rows = lax.broadcasted_iota(jnp.int32, (_BN, D), 0) + i * _BN
  hm = jnp.where(rows < N, h, 0.0)
  s1 = jnp.sum(hm, axis=0, keepdims=True)
  s2 = jnp.sum(hm * hm, axis=0, keepdims=True)
  acc = jnp.concatenate(
      [s1, s2, jnp.zeros((6, D), jnp.float32)], axis=0)

  @pl.when(i == 0)
  def _():
    st_ref[...] = acc

  @pl.when(i > 0)
  def _():
    st_ref[...] = st_ref[...] + acc


def _comb_call(p0, p1, nd, b2):
  return pl.pallas_call(
      _comb_body,
      grid=(_NBLK,),
      in_specs=[
          pl.BlockSpec((_BN, D), lambda i: (i, 0)),
          pl.BlockSpec((_BN, D), lambda i: (i, 0)),
          pl.BlockSpec((_BN, 1), lambda i: (i, 0)),
          pl.BlockSpec((1, D), lambda i: (0, 0)),
      ],
      out_specs=[
          pl.BlockSpec((_BN, D), lambda i: (i, 0)),
          pl.BlockSpec((8, D), lambda i: (0, 0)),
      ],
      out_shape=[
          jax.ShapeDtypeStruct((NP, D), jnp.float32),
          jax.ShapeDtypeStruct((8, D), jnp.float32),
      ],
  )(p0, p1, nd, b2)


def _bn_body(h2_ref, st_ref, g_ref, be_ref, x_ref, o_ref):
  mean = st_ref[0:1, :] * (1.0 / N)
  ex2 = st_ref[1:2, :] * (1.0 / N)
  var = ex2 - mean * mean
  inv = lax.rsqrt(var + 1e-5)
  hn = (h2_ref[...] - mean) * inv * g_ref[...] + be_ref[...]
  out = jnp.where(hn > 0, hn, jnp.exp(jnp.minimum(hn, 0.0)) - 1.0)
  o_ref[...] = x_ref[...] + out


def _bn_call(h2, stats, g2, be2, x):
  return pl.pallas_call(
      _bn_body,
      grid=(_NBLK,),
      in_specs=[
          pl.BlockSpec((_BN, D), lambda i: (i, 0)),
          pl.BlockSpec((8, D), lambda i: (0, 0)),
          pl.BlockSpec((1, D), lambda i: (0, 0)),
          pl.BlockSpec((1, D), lambda i: (0, 0)),
          pl.BlockSpec((_BN, D), lambda i: (i, 0)),
      ],
      out_specs=pl.BlockSpec((_BN, D), lambda i: (i, 0)),
      out_shape=jax.ShapeDtypeStruct((N, D), jnp.float32),
  )(h2, stats, g2, be2, x)


@jax.jit
def kernel(x, edge_index, W, b, gamma, beta):
  src = edge_index[0]
  dst = edge_index[1]
  pad = jnp.full((EP - E,), N, jnp.int32)
  src_p = jnp.concatenate([src, pad])
  dst_p = jnp.concatenate([dst, pad])
  sd_p = src_p | (dst_p << 16)
  degp = _deg_kernel(sd_p)
  norms = lax.rsqrt(jnp.maximum(degp.sum(axis=(0, 1)), 1.0))
  ns = norms[0][:, None]
  nd = norms[1][:, None]

  x_pad = jnp.concatenate(
      [x, jnp.zeros((NP - N, D), jnp.float32)], axis=0)
  h_scaled = _mm_call(x_pad, W, ns)

  parts = _gs_kernel(h_scaled, sd_p)

  h2, stats = _comb_call(parts[0], parts[1], nd, b.reshape(1, D))
  return _bn_call(h2, stats, gamma.reshape(1, D), beta.reshape(1, D), x)


# speedup vs baseline: 9.7755x; 1.0075x over previous
import functools

import jax
import jax.numpy as jnp
from jax import lax
from jax.experimental import pallas as pl
from jax.experimental.pallas import tpu as pltpu
from jax.experimental.pallas import tpu_sc as plsc

N = 10000
E = 320000
D = 128

NC = 2
NS = 16
NW = NC * NS

NP = 10240
EP = NW * NP
EPW = EP // NW
CB = 128
NCHUNK = EPW // CB
NCH_F = 130
NCH_S = 2 * NCHUNK - NCH_F
FAST_CID = 0
DGF = 1088
DGS = 2 * (EPW // 16) - DGF
AGR = 10008
AGT = 632
ROWS = NP // 128
NPT = NP // NS

_mesh = plsc.VectorSubcoreMesh(core_axis_name="c", subcore_axis_name="s")


@functools.partial(
    pl.kernel,
    out_type=jax.ShapeDtypeStruct((NC, NS, 2, NP), jnp.float32),
    mesh=_mesh,
    compiler_params=pltpu.CompilerParams(needs_layout_passes=False),
    scratch_types=[
        pltpu.VMEM((DGF * 16,), jnp.int32),
        pltpu.VMEM((NP,), jnp.float32),
        pltpu.VMEM((NP,), jnp.float32),
    ],
)
def _deg_kernel(sd_hbm, out_hbm, sdidx, hist_s, hist_d):
  cid = lax.axis_index("c")
  sid = lax.axis_index("s")

  is_fast = cid == FAST_CID
  ngr = jnp.where(is_fast, DGF, DGS)
  base_el = jnp.where(is_fast, sid * DGF * 16,
                      NS * DGF * 16 + sid * DGS * 16)
  ld_base = jnp.minimum(base_el, EP - DGF * 16)
  off = base_el - ld_base

  zeros16 = jnp.zeros((16,), jnp.float32)
  ones16 = jnp.ones((16,), jnp.float32)

  def zbody(i, carry):
    hist_s[pl.ds(i * 16, 16)] = zeros16
    hist_d[pl.ds(i * 16, 16)] = zeros16
    return carry

  lax.fori_loop(0, NP // 16, zbody, 0)

  pltpu.sync_copy(sd_hbm.at[pl.ds(ld_base, DGF * 16)], sdidx)

  def hbody(i, carry):
    v = sdidx[pl.ds(off + i * 16, 16)]
    plsc.addupdate_scatter(hist_s, [v & 0xFFFF], ones16)
    plsc.addupdate_scatter(hist_d, [v >> 16], ones16)
    return carry

  lax.fori_loop(0, ngr, hbody, 0)

  pltpu.sync_copy(hist_s, out_hbm.at[cid, sid, 0])
  pltpu.sync_copy(hist_d, out_hbm.at[cid, sid, 1])


@functools.partial(
    pl.kernel,
    out_type=jax.ShapeDtypeStruct((NC, NP, D), jnp.float32),
    mesh=_mesh,
    scratch_types=[
        pltpu.VMEM((NCH_F * CB,), jnp.int32),
        pltpu.VMEM((NCH_F * CB,), jnp.int32),
        pltpu.VMEM((CB, D), jnp.float32),
        pltpu.VMEM_SHARED((AGR, D), jnp.float32),
        pltpu.SemaphoreType.DMA,
    ],
)
def _gs_kernel(h_hbm, sd_hbm, out_hbm,
               sidx, didx, buf0, agg_sh, sem0):
  cid = lax.axis_index("c")
  sid = lax.axis_index("s")

  is_fast = cid == FAST_CID
  nch = jnp.where(is_fast, NCH_F, NCH_S)
  base = jnp.where(is_fast, sid * NCH_F, NS * NCH_F + sid * NCH_S)

  pltpu.sync_copy(h_hbm.at[pl.ds(NP - CB, CB)], buf0)
  for k in range(4):
    pltpu.sync_copy(buf0, agg_sh.at[pl.ds(sid * AGT + k * CB, CB)])

  @pl.when(sid < NS - 1)
  def _():
    pltpu.sync_copy(buf0.at[pl.ds(0, 120)],
                    agg_sh.at[pl.ds(sid * AGT + 512, 120)])

  @pl.when(sid == NS - 1)
  def _():
    pltpu.sync_copy(buf0.at[pl.ds(0, 16)],
                    agg_sh.at[pl.ds((NS - 1) * AGT + 512, 16)])
  ld_base = jnp.minimum(base * CB, EP - NCH_F * CB)
  off = base * CB - ld_base
  pltpu.sync_copy(sd_hbm.at[pl.ds(ld_base, NCH_F * CB)], sidx)

  def ubody(i, carry):
    v = sidx[pl.ds(i * 16, 16)]
    sidx[pl.ds(i * 16, 16)] = v & 0xFFFF
    didx[pl.ds(i * 16, 16)] = v >> 16
    return carry

  lax.fori_loop(0, NCH_F * CB // 16, ubody, 0)
  plsc.subcore_barrier()

  def body(j, carry):
    s = off + j * CB
    pltpu.async_copy(h_hbm.at[sidx.at[pl.ds(s, CB)]], buf0, sem0).wait()
    pltpu.sync_copy(buf0, agg_sh.at[didx.at[pl.ds(s, CB)]], add=True)
    return carry

  lax.fori_loop(0, nch, body, 0)

  plsc.subcore_barrier()

  @pl.when(sid < NS - 1)
  def _():
    pltpu.sync_copy(agg_sh.at[pl.ds(sid * AGT, AGT)],
                    out_hbm.at[cid, pl.ds(sid * AGT, AGT)])

  @pl.when(sid == NS - 1)
  def _():
    pltpu.sync_copy(agg_sh.at[pl.ds((NS - 1) * AGT, 528)],
                    out_hbm.at[cid, pl.ds((NS - 1) * AGT, 528)])


def _nrm_body(dp_ref, o_ref):
  x = dp_ref[...]
  row = lax.broadcasted_iota(jnp.int32, (2 * NW, _BN), 0)
  even = (row & 1) == 0
  s_src = jnp.sum(jnp.where(even, x, 0.0), axis=0, keepdims=True)
  s_dst = jnp.sum(jnp.where(even, 0.0, x), axis=0, keepdims=True)
  deg = jnp.concatenate([s_src, s_dst], axis=0)
  o_ref[...] = lax.rsqrt(jnp.maximum(deg, 1.0))


def _nrm_call(degp64):
  return pl.pallas_call(
      _nrm_body,
      grid=(NP // _BN,),
      in_specs=[pl.BlockSpec((2 * NW, _BN), lambda i: (0, i))],
      out_specs=pl.BlockSpec((2, _BN), lambda i: (0, i)),
      out_shape=jax.ShapeDtypeStruct((2, NP), jnp.float32),
  )(degp64)


def _mm_body(x_ref, w_ref, ns_ref, o_ref):
  h = jnp.dot(x_ref[...], w_ref[...], preferred_element_type=jnp.float32)
  o_ref[...] = h * ns_ref[...]


_BN = 1024
_NBLK = NP // _BN


def _mm_call(x_pad, W, ns):
  return pl.pallas_call(
      _mm_body,
      grid=(_NBLK,),
      in_specs=[
          pl.BlockSpec((_BN, D), lambda i: (i, 0)),
          pl.BlockSpec((D, D), lambda i: (0, 0)),
          pl.BlockSpec((_BN, 1), lambda i: (i, 0)),
      ],
      out_specs=pl.BlockSpec((_BN, D), lambda i: (i, 0)),
      out_shape=jax.ShapeDtypeStruct((NP, D), jnp.float32),
  )(x_pad, W, ns)


def _comb_body(p0_ref, p1_ref, nd_ref, b_ref, h2_ref, st_ref):
  i = pl.program_id(0)
  h = (p0_ref[...] + p1_ref[...]) * nd_ref[...] + b_ref[...]
  h2_ref[...] = h
  rows = lax.broadcasted_iota(jnp.int32, (_BN, D), 0) + i * _BN
  hm = jnp.where(rows < N, h, 0.0)
  s1 = jnp.sum(hm, axis=0, keepdims=True)
  s2 = jnp.sum(hm * hm, axis=0, keepdims=True)
  acc = jnp.concatenate(
      [s1, s2, jnp.zeros((6, D), jnp.float32)], axis=0)

  @pl.when(i == 0)
  def _():
    st_ref[...] = acc

  @pl.when(i > 0)
  def _():
    st_ref[...] = st_ref[...] + acc


def _comb_call(p0, p1, nd, b2):
  return pl.pallas_call(
      _comb_body,
      grid=(_NBLK,),
      in_specs=[
          pl.BlockSpec((_BN, D), lambda i: (i, 0)),
          pl.BlockSpec((_BN, D), lambda i: (i, 0)),
          pl.BlockSpec((_BN, 1), lambda i: (i, 0)),
          pl.BlockSpec((1, D), lambda i: (0, 0)),
      ],
      out_specs=[
          pl.BlockSpec((_BN, D), lambda i: (i, 0)),
          pl.BlockSpec((8, D), lambda i: (0, 0)),
      ],
      out_shape=[
          jax.ShapeDtypeStruct((NP, D), jnp.float32),
          jax.ShapeDtypeStruct((8, D), jnp.float32),
      ],
  )(p0, p1, nd, b2)


def _bn_body(h2_ref, st_ref, g_ref, be_ref, x_ref, o_ref):
  mean = st_ref[0:1, :] * (1.0 / N)
  ex2 = st_ref[1:2, :] * (1.0 / N)
  var = ex2 - mean * mean
  inv = lax.rsqrt(var + 1e-5)
  hn = (h2_ref[...] - mean) * inv * g_ref[...] + be_ref[...]
  out = jnp.where(hn > 0, hn, jnp.exp(jnp.minimum(hn, 0.0)) - 1.0)
  o_ref[...] = x_ref[...] + out


def _bn_call(h2, stats, g2, be2, x):
  return pl.pallas_call(
      _bn_body,
      grid=(_NBLK,),
      in_specs=[
          pl.BlockSpec((_BN, D), lambda i: (i, 0)),
          pl.BlockSpec((8, D), lambda i: (0, 0)),
          pl.BlockSpec((1, D), lambda i: (0, 0)),
          pl.BlockSpec((1, D), lambda i: (0, 0)),
          pl.BlockSpec((_BN, D), lambda i: (i, 0)),
      ],
      out_specs=pl.BlockSpec((_BN, D), lambda i: (i, 0)),
      out_shape=jax.ShapeDtypeStruct((N, D), jnp.float32),
  )(h2, stats, g2, be2, x)


@jax.jit
def kernel(x, edge_index, W, b, gamma, beta):
  src = edge_index[0]
  dst = edge_index[1]
  pad = jnp.full((EP - E,), N, jnp.int32)
  src_p = jnp.concatenate([src, pad])
  dst_p = jnp.concatenate([dst, pad])
  sd_p = src_p | (dst_p << 16)
  degp = _deg_kernel(sd_p)
  norms = lax.rsqrt(jnp.maximum(degp.sum(axis=(0, 1)), 1.0))
  ns = norms[0][:, None]
  nd = norms[1][:, None]

  x_pad = jnp.concatenate(
      [x, jnp.zeros((NP - N, D), jnp.float32)], axis=0)
  h_scaled = _mm_call(x_pad, W, ns)

  parts = _gs_kernel(h_scaled, sd_p)

  h2, stats = _comb_call(parts[0], parts[1], nd, b.reshape(1, D))
  return _bn_call(h2, stats, gamma.reshape(1, D), beta.reshape(1, D), x)
